# baseline (speedup 1.0000x reference)
import functools

import jax
import jax.numpy as jnp
from jax.experimental import pallas as pl
from jax.experimental.pallas import tpu as pltpu


def _cell_kernel(
    x_ref,
    cc_ref,
    ch_ref,
    w_x_ref,
    w_iouh_ref,
    w_fh_ref,
    b_ref,
    c_out_ref,
    h_out_ref,
    *,
    mem_dim: int,
    num_children: int,
):
    m = mem_dim
    cd = w_x_ref.dtype

    x = x_ref[...].astype(cd)
    z = jnp.dot(x, w_x_ref[...], preferred_element_type=jnp.float32)
    z = z + b_ref[...]

    cc = cc_ref[...]
    s = cc[0]
    for k in range(1, num_children):
        s = s + cc[k]
    iou = z[:, : 3 * m] + jnp.dot(
        s.astype(cd), w_iouh_ref[...], preferred_element_type=jnp.float32
    )

    i_g = jax.nn.sigmoid(iou[:, 0 * m:1 * m])
    o_g = jax.nn.sigmoid(iou[:, 1 * m:2 * m])
    u_g = jnp.tanh(iou[:, 2 * m:3 * m])
    fx = z[:, 3 * m:4 * m]

    c = i_g * u_g
    ch = ch_ref[...].astype(cd)
    for k in range(num_children):
        fh = jnp.dot(ch[k], w_fh_ref[...], preferred_element_type=jnp.float32)
        c = c + jax.nn.sigmoid(fx * fh)

    h = o_g * jnp.tanh(c)
    c_out_ref[...] = c
    h_out_ref[...] = h.astype(h_out_ref.dtype)


def kernel(x, child_c, child_h, w_x, w_iouh, w_fh, b_all):
    N, in_dim = x.shape
    K = int(child_c.shape[0])
    mem = int(w_fh.shape[0])

    tile = min(2048, N)
    if N % tile != 0:
        tile = N
    num_tiles = N // tile
    if num_tiles % 2 == 0:
        grid = (2, num_tiles // 2)
        semantics = ("parallel", "arbitrary")

        def row(i, j):
            return i * (num_tiles // 2) + j

        in_specs = [
            pl.BlockSpec((tile, in_dim), lambda i, j: (row(i, j), 0)),
            pl.BlockSpec((K, tile, mem), lambda i, j: (0, row(i, j), 0)),
            pl.BlockSpec((K, tile, mem), lambda i, j: (0, row(i, j), 0)),
            pl.BlockSpec((in_dim, 4 * mem), lambda i, j: (0, 0)),
            pl.BlockSpec((mem, 3 * mem), lambda i, j: (0, 0)),
            pl.BlockSpec((mem, mem), lambda i, j: (0, 0)),
            pl.BlockSpec((1, 4 * mem), lambda i, j: (0, 0)),
        ]
        out_specs = (
            pl.BlockSpec((tile, mem), lambda i, j: (row(i, j), 0)),
            pl.BlockSpec((tile, mem), lambda i, j: (row(i, j), 0)),
        )
    else:
        grid = (num_tiles,)
        semantics = ("parallel",)
        in_specs = [
            pl.BlockSpec((tile, in_dim), lambda i: (i, 0)),
            pl.BlockSpec((K, tile, mem), lambda i: (0, i, 0)),
            pl.BlockSpec((K, tile, mem), lambda i: (0, i, 0)),
            pl.BlockSpec((in_dim, 4 * mem), lambda i: (0, 0)),
            pl.BlockSpec((mem, 3 * mem), lambda i: (0, 0)),
            pl.BlockSpec((mem, mem), lambda i: (0, 0)),
            pl.BlockSpec((1, 4 * mem), lambda i: (0, 0)),
        ]
        out_specs = (
            pl.BlockSpec((tile, mem), lambda i: (i, 0)),
            pl.BlockSpec((tile, mem), lambda i: (i, 0)),
        )

    blk_bytes = (
        tile * in_dim * 4
        + 2 * K * tile * mem * 4
        + tile * mem * 4
        + tile * mem * 2
    )
    w_bytes = (w_x.size + w_iouh.size + w_fh.size) * 2 + b_all.size * 4
    vmem_limit = int(min(max(3 * blk_bytes + w_bytes, 32 << 20), 64 << 20))

    flops = 2 * N * in_dim * 4 * mem + 2 * N * mem * 3 * mem + 2 * K * N * mem * mem
    bytes_accessed = (
        x.size * 4 + child_c.size * 4 + child_h.size * 4
        + w_bytes + N * mem * (4 + 2)
    )

    kernel_fn = functools.partial(_cell_kernel, mem_dim=mem, num_children=K)
    c_out, h_out = pl.pallas_call(
        kernel_fn,
        out_shape=(
            jax.ShapeDtypeStruct((N, mem), jnp.float32),
            jax.ShapeDtypeStruct((N, mem), jnp.bfloat16),
        ),
        grid=grid,
        in_specs=in_specs,
        out_specs=out_specs,
        compiler_params=pltpu.CompilerParams(
            dimension_semantics=semantics,
            vmem_limit_bytes=vmem_limit,
        ),
        cost_estimate=pl.CostEstimate(
            flops=flops,
            transcendentals=(4 + K) * N * mem,
            bytes_accessed=bytes_accessed,
        ),
    )(x, child_c, child_h, w_x, w_iouh, w_fh, b_all)
    return c_out, h_out


# speedup vs baseline: 1.0281x; 1.0281x over previous
import functools

import jax
import jax.numpy as jnp
from jax.experimental import pallas as pl
from jax.experimental.pallas import tpu as pltpu


def _cell_kernel(
    x_ref,
    cc_ref,
    ch_ref,
    w_x_ref,
    w_iouh_ref,
    w_fh_ref,
    b_ref,
    c_out_ref,
    h_out_ref,
    *,
    mem_dim: int,
    num_children: int,
):
    m = mem_dim
    cd = w_x_ref.dtype

    x = x_ref[...].astype(cd)
    z = jnp.dot(x, w_x_ref[...], preferred_element_type=jnp.float32)
    z = z + b_ref[...]

    cc = cc_ref[...]
    s = cc[0]
    for k in range(1, num_children):
        s = s + cc[k]
    iou = z[:, : 3 * m] + jnp.dot(
        s.astype(cd), w_iouh_ref[...], preferred_element_type=jnp.float32
    )

    i_g = jax.nn.sigmoid(iou[:, 0 * m:1 * m])
    o_g = jax.nn.sigmoid(iou[:, 1 * m:2 * m])
    u_g = jnp.tanh(iou[:, 2 * m:3 * m])
    fx = z[:, 3 * m:4 * m]

    c = i_g * u_g
    ch = ch_ref[...].astype(cd)
    for k in range(num_children):
        fh = jnp.dot(ch[k], w_fh_ref[...], preferred_element_type=jnp.float32)
        c = c + jax.nn.sigmoid(fx * fh)

    h = o_g * jnp.tanh(c)
    c_out_ref[...] = c
    h_out_ref[...] = h.astype(h_out_ref.dtype)


def kernel(x, child_c, child_h, w_x, w_iouh, w_fh, b_all):
    N, in_dim = x.shape
    K = int(child_c.shape[0])
    mem = int(w_fh.shape[0])

    tile = min(4096, N)
    if N % tile != 0:
        tile = N
    num_tiles = N // tile
    if num_tiles % 2 == 0:
        grid = (2, num_tiles // 2)
        semantics = ("parallel", "arbitrary")

        def row(i, j):
            return i * (num_tiles // 2) + j

        in_specs = [
            pl.BlockSpec((tile, in_dim), lambda i, j: (row(i, j), 0)),
            pl.BlockSpec((K, tile, mem), lambda i, j: (0, row(i, j), 0)),
            pl.BlockSpec((K, tile, mem), lambda i, j: (0, row(i, j), 0)),
            pl.BlockSpec((in_dim, 4 * mem), lambda i, j: (0, 0)),
            pl.BlockSpec((mem, 3 * mem), lambda i, j: (0, 0)),
            pl.BlockSpec((mem, mem), lambda i, j: (0, 0)),
            pl.BlockSpec((1, 4 * mem), lambda i, j: (0, 0)),
        ]
        out_specs = (
            pl.BlockSpec((tile, mem), lambda i, j: (row(i, j), 0)),
            pl.BlockSpec((tile, mem), lambda i, j: (row(i, j), 0)),
        )
    else:
        grid = (num_tiles,)
        semantics = ("parallel",)
        in_specs = [
            pl.BlockSpec((tile, in_dim), lambda i: (i, 0)),
            pl.BlockSpec((K, tile, mem), lambda i: (0, i, 0)),
            pl.BlockSpec((K, tile, mem), lambda i: (0, i, 0)),
            pl.BlockSpec((in_dim, 4 * mem), lambda i: (0, 0)),
            pl.BlockSpec((mem, 3 * mem), lambda i: (0, 0)),
            pl.BlockSpec((mem, mem), lambda i: (0, 0)),
            pl.BlockSpec((1, 4 * mem), lambda i: (0, 0)),
        ]
        out_specs = (
            pl.BlockSpec((tile, mem), lambda i: (i, 0)),
            pl.BlockSpec((tile, mem), lambda i: (i, 0)),
        )

    blk_bytes = (
        tile * in_dim * 4
        + 2 * K * tile * mem * 4
        + tile * mem * 4
        + tile * mem * 2
    )
    w_bytes = (w_x.size + w_iouh.size + w_fh.size) * 2 + b_all.size * 4
    vmem_limit = int(min(max(3 * blk_bytes + w_bytes, 32 << 20), 64 << 20))

    flops = 2 * N * in_dim * 4 * mem + 2 * N * mem * 3 * mem + 2 * K * N * mem * mem
    bytes_accessed = (
        x.size * 4 + child_c.size * 4 + child_h.size * 4
        + w_bytes + N * mem * (4 + 2)
    )

    kernel_fn = functools.partial(_cell_kernel, mem_dim=mem, num_children=K)
    c_out, h_out = pl.pallas_call(
        kernel_fn,
        out_shape=(
            jax.ShapeDtypeStruct((N, mem), jnp.float32),
            jax.ShapeDtypeStruct((N, mem), jnp.bfloat16),
        ),
        grid=grid,
        in_specs=in_specs,
        out_specs=out_specs,
        compiler_params=pltpu.CompilerParams(
            dimension_semantics=semantics,
            vmem_limit_bytes=vmem_limit,
        ),
        cost_estimate=pl.CostEstimate(
            flops=flops,
            transcendentals=(4 + K) * N * mem,
            bytes_accessed=bytes_accessed,
        ),
    )(x, child_c, child_h, w_x, w_iouh, w_fh, b_all)
    return c_out, h_out
